# Initial kernel scaffold; baseline (speedup 1.0000x reference)
import jax
import jax.numpy as jnp
from jax.experimental import pallas as pl


def kernel(features, feat_t_g):
    raise NotImplementedError("write your pallas kernel here")


# speedup vs baseline: 4.8476x; 4.8476x over previous
import functools

import jax
import jax.numpy as jnp
from jax.experimental import pallas as pl
from jax.experimental.pallas import tpu as pltpu

_N = 256
_K = 5
_INV_TEMPERATURE = 1.0 / 0.07


def _nncon_loss_kernel(features_ref, feat_t_g_ref, out_ref):
    g = feat_t_g_ref[...]
    f = features_ref[...]

    sim = jax.lax.dot_general(
        g, g, (((1,), (1,)), ((), ())), preferred_element_type=jnp.float32
    )

    col = jax.lax.broadcasted_iota(jnp.int32, (_N, _N), 1)

    work = sim
    mask = jnp.zeros((_N, _N), dtype=jnp.float32)
    for _ in range(_K):
        row_max = jnp.max(work, axis=1, keepdims=True)
        at_max = work >= row_max
        first = jnp.min(jnp.where(at_max, col, _N), axis=1, keepdims=True)
        sel = col == first
        mask = mask + sel.astype(jnp.float32)
        work = jnp.where(sel, -jnp.inf, work)

    row = jax.lax.broadcasted_iota(jnp.int32, (_N, _N), 0)
    off_diag = (row != col).astype(jnp.float32)
    mask = mask * off_diag

    adc = (
        jax.lax.dot_general(
            f, f, (((1,), (1,)), ((), ())), preferred_element_type=jnp.float32
        )
        * _INV_TEMPERATURE
    )
    logits_max = jnp.max(adc, axis=1, keepdims=True)
    logits = adc - logits_max

    exp_sum = jnp.sum(jnp.exp(logits) * off_diag, axis=1, keepdims=True)
    log_prob = logits - jnp.log(exp_sum)

    msum = jnp.sum(mask, axis=1)
    denom = jnp.where(msum == 0.0, 1.0, msum)
    mean_log_prob_pos = jnp.sum(mask * log_prob, axis=1) / denom

    out_ref[...] = (-jnp.sum(mean_log_prob_pos) / _N).reshape(1, 1)


@jax.jit
def kernel(features, feat_t_g):
    out = pl.pallas_call(
        _nncon_loss_kernel,
        out_shape=jax.ShapeDtypeStruct((1, 1), jnp.float32),
    )(features, feat_t_g)
    return out[0, 0]
